# Initial kernel scaffold; baseline (speedup 1.0000x reference)
import jax
import jax.numpy as jnp
from jax.experimental import pallas as pl


def kernel(x, edge_index, edge_attr, a1_w, a2_w, ae_w, W1, b1, gamma, beta, W2, b2):
    raise NotImplementedError("write your pallas kernel here")


# speedup vs baseline: 2.1131x; 2.1131x over previous
import functools
import math

import jax
import jax.numpy as jnp
from jax.experimental import pallas as pl
from jax.experimental.pallas import tpu as pltpu

N, E, D, DE, H, OH = 10000, 160000, 256, 16, 4, 64
HP = 8
HOH = H * OH
RT = 1000
ET = 2000


def _a1_body(x_ref, a2t_ref, w1x_ref, w1o_ref, p2_ref, xw_ref, y0_ref, y1_ref):
    xt = x_ref[...]
    p2_ref[...] = jnp.exp(jnp.dot(xt, a2t_ref[...],
                                  preferred_element_type=jnp.float32)
                          * (1.0 / math.sqrt(OH)))
    xw_ref[...] = jnp.dot(xt, w1x_ref[...], preferred_element_type=jnp.float32)
    y = jnp.dot(xt, w1o_ref[...], preferred_element_type=jnp.float32)
    y0_ref[...] = y[:, :128]
    y1_ref[...] = y[:, 128:]


def _a1(x, a2t, w1x, w1o):
    return pl.pallas_call(
        _a1_body,
        grid=(N // RT,),
        in_specs=[
            pl.BlockSpec((RT, D), lambda i: (i, 0)),
            pl.BlockSpec((D, HP), lambda i: (0, 0)),
            pl.BlockSpec((D, HOH), lambda i: (0, 0)),
            pl.BlockSpec((D, HOH), lambda i: (0, 0)),
        ],
        out_specs=[
            pl.BlockSpec((RT, HP), lambda i: (i, 0)),
            pl.BlockSpec((RT, HOH), lambda i: (i, 0)),
            pl.BlockSpec((RT, 128), lambda i: (i, 0)),
            pl.BlockSpec((RT, 128), lambda i: (i, 0)),
        ],
        out_shape=[
            jax.ShapeDtypeStruct((N, HP), jnp.float32),
            jax.ShapeDtypeStruct((N, HOH), jnp.float32),
            jax.ShapeDtypeStruct((N, 128), jnp.float32),
            jax.ShapeDtypeStruct((N, 128), jnp.float32),
        ],
    )(x, a2t, w1x, w1o)


def _a2_body(ea_ref, aet_ref, w1e_ref, pe_ref, ye0_ref, ye1_ref):
    ea = ea_ref[...]
    pe_ref[...] = jnp.exp(jnp.dot(ea, aet_ref[...],
                                  preferred_element_type=jnp.float32)
                          * (1.0 / math.sqrt(OH)))
    ye = jnp.dot(ea, w1e_ref[...], preferred_element_type=jnp.float32)
    ye0_ref[...] = ye[:, :128]
    ye1_ref[...] = ye[:, 128:]


def _a2(ea, aet, w1e):
    return pl.pallas_call(
        _a2_body,
        grid=(E // ET,),
        in_specs=[
            pl.BlockSpec((ET, DE), lambda i: (i, 0)),
            pl.BlockSpec((DE, HP), lambda i: (0, 0)),
            pl.BlockSpec((DE, HOH), lambda i: (0, 0)),
        ],
        out_specs=[
            pl.BlockSpec((ET, HP), lambda i: (i, 0)),
            pl.BlockSpec((ET, 128), lambda i: (i, 0)),
            pl.BlockSpec((ET, 128), lambda i: (i, 0)),
        ],
        out_shape=[
            jax.ShapeDtypeStruct((E, HP), jnp.float32),
            jax.ShapeDtypeStruct((E, 128), jnp.float32),
            jax.ShapeDtypeStruct((E, 128), jnp.float32),
        ],
    )(ea, aet, w1e)


def _c1_body(xw_ref, z0_ref, z1_ref, b1_ref, h_ref, st_ref):
    ht = jax.nn.relu(xw_ref[...]
                     + jnp.concatenate([z0_ref[...], z1_ref[...]], axis=1)
                     + b1_ref[...])
    h_ref[...] = ht
    st = jnp.stack([jnp.sum(ht, axis=0), jnp.sum(ht * ht, axis=0)])
    i = pl.program_id(0)

    @pl.when(i == 0)
    def _():
        st_ref[...] = st

    @pl.when(i > 0)
    def _():
        st_ref[...] += st


def _c1(xw, z0, z1, b1c):
    return pl.pallas_call(
        _c1_body,
        grid=(N // RT,),
        in_specs=[
            pl.BlockSpec((RT, HOH), lambda i: (i, 0)),
            pl.BlockSpec((RT, 128), lambda i: (i, 0)),
            pl.BlockSpec((RT, 128), lambda i: (i, 0)),
            pl.BlockSpec((1, HOH), lambda i: (0, 0)),
        ],
        out_specs=[
            pl.BlockSpec((RT, HOH), lambda i: (i, 0)),
            pl.BlockSpec((2, HOH), lambda i: (0, 0)),
        ],
        out_shape=[
            jax.ShapeDtypeStruct((N, HOH), jnp.float32),
            jax.ShapeDtypeStruct((2, HOH), jnp.float32),
        ],
    )(xw, z0, z1, b1c)


def _c2_body(h_ref, st_ref, g_ref, bt_ref, b2_ref, w2_ref, out_ref):
    st = st_ref[...]
    mu = st[0:1] * (1.0 / N)
    var = st[1:2] * (1.0 / N) - mu * mu
    scale = g_ref[...] * jax.lax.rsqrt(var + 1e-5)
    w2eff = w2_ref[...] * scale.reshape(HOH, 1)
    bias = (jnp.dot(bt_ref[...] - mu * scale, w2_ref[...],
                    preferred_element_type=jnp.float32) + b2_ref[...])
    out_ref[...] = (jnp.dot(h_ref[...], w2eff,
                            preferred_element_type=jnp.float32) + bias)


def _c2(h, st, gc, bc, b2c, w2bd):
    return pl.pallas_call(
        _c2_body,
        grid=(N // RT,),
        in_specs=[
            pl.BlockSpec((RT, HOH), lambda i: (i, 0)),
            pl.BlockSpec((2, HOH), lambda i: (0, 0)),
            pl.BlockSpec((1, HOH), lambda i: (0, 0)),
            pl.BlockSpec((1, HOH), lambda i: (0, 0)),
            pl.BlockSpec((1, HOH), lambda i: (0, 0)),
            pl.BlockSpec((HOH, HOH), lambda i: (0, 0)),
        ],
        out_specs=pl.BlockSpec((RT, HOH), lambda i: (i, 0)),
        out_shape=jax.ShapeDtypeStruct((N, HOH), jnp.float32),
    )(h, st, gc, bc, b2c, w2bd)


def kernel(x, edge_index, edge_attr, a1_w, a2_w, ae_w, W1, b1, gamma, beta, W2, b2):
    row, col = edge_index[0], edge_index[1]
    a2t = jnp.zeros((D, HP), jnp.float32).at[:, :H].set(a2_w.T)
    aet = jnp.zeros((DE, HP), jnp.float32).at[:, :H].set(ae_w.T)
    w1x = W1[:, :D].transpose(1, 0, 2).reshape(D, HOH)
    w1o = W1[:, D:2 * D].transpose(1, 0, 2).reshape(D, HOH)
    w1e = W1[:, 2 * D:].transpose(1, 0, 2).reshape(DE, HOH)
    b1c = b1.reshape(1, HOH)
    gc = gamma.reshape(1, HOH)
    bc = beta.reshape(1, HOH)
    b2c = b2.reshape(1, HOH)
    w2bd = jax.scipy.linalg.block_diag(*[W2[i] for i in range(H)])

    p2, xw, y0, y1 = _a1(x, a2t, w1x, w1o)
    pe, ye0, ye1 = _a2(edge_attr, aet, w1e)

    ev = p2[col] * pe
    s = jax.ops.segment_sum(ev, row, num_segments=N)
    rinv = 1.0 / (s + 1e-16)
    w = ev * rinv[row]
    w0 = jnp.repeat(w[:, :2], OH, axis=1)
    w1r = jnp.repeat(w[:, 2:4], OH, axis=1)
    v0 = w0 * (y0[col] + ye0)
    v1 = w1r * (y1[col] + ye1)
    z0 = jax.ops.segment_sum(v0, row, num_segments=N)
    z1 = jax.ops.segment_sum(v1, row, num_segments=N)

    h, st = _c1(xw, z0, z1, b1c)
    return _c2(h, st, gc, bc, b2c, w2bd)


# speedup vs baseline: 6.1728x; 2.9212x over previous
import functools
import math

import jax
import jax.numpy as jnp
from jax import lax
from jax.experimental import pallas as pl
from jax.experimental.pallas import tpu as pltpu
from jax.experimental.pallas import tpu_sc as plsc

N, E, D, DE, H, OH = 10000, 160000, 256, 16, 4, 64
HP = 16
HOH = H * OH
RT = 1000
ET = 2000

NP = 10240
NR = NP // 16
CM = 128
EP = 161792
EPT = EP // 16

_mesh = plsc.VectorSubcoreMesh(core_axis_name="c", subcore_axis_name="s",
                               num_cores=2, num_subcores=16)


def _a1_body(x_ref, a2t_ref, w1x_ref, w1o_ref, p2_ref, xw_ref, y0_ref, y1_ref):
    xt = x_ref[...]
    p2_ref[...] = jnp.exp(jnp.dot(xt, a2t_ref[...],
                                  preferred_element_type=jnp.float32)
                          * (1.0 / math.sqrt(OH)))
    xw_ref[...] = jnp.dot(xt, w1x_ref[...], preferred_element_type=jnp.float32)
    y = jnp.dot(xt, w1o_ref[...], preferred_element_type=jnp.float32)
    y0_ref[...] = y[:, :128]
    y1_ref[...] = y[:, 128:]


def _a1(x, a2t, w1x, w1o):
    return pl.pallas_call(
        _a1_body,
        grid=(N // RT,),
        in_specs=[
            pl.BlockSpec((RT, D), lambda i: (i, 0)),
            pl.BlockSpec((D, HP), lambda i: (0, 0)),
            pl.BlockSpec((D, HOH), lambda i: (0, 0)),
            pl.BlockSpec((D, HOH), lambda i: (0, 0)),
        ],
        out_specs=[
            pl.BlockSpec((RT, HP), lambda i: (i, 0)),
            pl.BlockSpec((RT, HOH), lambda i: (i, 0)),
            pl.BlockSpec((RT, 128), lambda i: (i, 0)),
            pl.BlockSpec((RT, 128), lambda i: (i, 0)),
        ],
        out_shape=[
            jax.ShapeDtypeStruct((N, HP), jnp.float32),
            jax.ShapeDtypeStruct((N, HOH), jnp.float32),
            jax.ShapeDtypeStruct((N, 128), jnp.float32),
            jax.ShapeDtypeStruct((N, 128), jnp.float32),
        ],
    )(x, a2t, w1x, w1o)


def _a2_body(ea_ref, aet_ref, w1e_ref, pe_ref, ye0_ref, ye1_ref):
    ea = ea_ref[...]
    pe_ref[...] = jnp.exp(jnp.dot(ea, aet_ref[...],
                                  preferred_element_type=jnp.float32)
                          * (1.0 / math.sqrt(OH)))
    ye = jnp.dot(ea, w1e_ref[...], preferred_element_type=jnp.float32)
    ye0_ref[...] = ye[:, :128]
    ye1_ref[...] = ye[:, 128:]


def _a2(ea, aet, w1e):
    return pl.pallas_call(
        _a2_body,
        grid=(E // ET,),
        in_specs=[
            pl.BlockSpec((ET, DE), lambda i: (i, 0)),
            pl.BlockSpec((DE, HP), lambda i: (0, 0)),
            pl.BlockSpec((DE, HOH), lambda i: (0, 0)),
        ],
        out_specs=[
            pl.BlockSpec((ET, HP), lambda i: (i, 0)),
            pl.BlockSpec((ET, 128), lambda i: (i, 0)),
            pl.BlockSpec((ET, 128), lambda i: (i, 0)),
        ],
        out_shape=[
            jax.ShapeDtypeStruct((E, HP), jnp.float32),
            jax.ShapeDtypeStruct((E, 128), jnp.float32),
            jax.ShapeDtypeStruct((E, 128), jnp.float32),
        ],
    )(ea, aet, w1e)


@functools.partial(
    pl.kernel, mesh=_mesh,
    out_type=[jax.ShapeDtypeStruct((NP,), jnp.float32),
              jax.ShapeDtypeStruct((NP,), jnp.float32),
              jax.ShapeDtypeStruct((NP,), jnp.float32),
              jax.ShapeDtypeStruct((NP,), jnp.float32),
              jax.ShapeDtypeStruct((NP, 128), jnp.float32),
              jax.ShapeDtypeStruct((NP, 128), jnp.float32)],
    scratch_types=[pltpu.VMEM((CM,), jnp.int32),
                   pltpu.VMEM((CM,), jnp.int32),
                   pltpu.VMEM((CM,), jnp.float32),
                   pltpu.VMEM((CM,), jnp.float32),
                   pltpu.VMEM((CM,), jnp.float32),
                   pltpu.VMEM((CM,), jnp.float32),
                   pltpu.VMEM((CM, 128), jnp.float32),
                   pltpu.VMEM((CM, 128), jnp.float32),
                   pltpu.VMEM_SHARED((NP,), jnp.float32),
                   pltpu.VMEM_SHARED((NP,), jnp.float32),
                   pltpu.VMEM_SHARED((NP, 128), jnp.float32),
                   pltpu.SemaphoreType.DMA])
def _sc(row_h, col_h, p20_h, p21_h, p22_h, p23_h, pe0_h, pe1_h, pe2_h, pe3_h,
        y0_h, y1_h, ye0_h, ye1_h, zz1_h, zz128_h,
        s0_h, s1_h, s2_h, s3_h, z0_h, z1_h,
        colv, rowv, g0, g1, w0b, w1b, ygb, yeb, sa_sh, sb_sh, z_sh, sem):
    c = lax.axis_index("c")
    sid = lax.axis_index("s")
    pltpu.sync_copy(zz1_h.at[pl.ds(sid * NR, NR)], sa_sh.at[pl.ds(sid * NR, NR)])
    pltpu.sync_copy(zz1_h.at[pl.ds(sid * NR, NR)], sb_sh.at[pl.ds(sid * NR, NR)])
    pltpu.sync_copy(zz128_h.at[pl.ds(sid * NR, NR)], z_sh.at[pl.ds(sid * NR, NR)])
    plsc.subcore_barrier()

    def run(p2a_h, p2b_h, pea_h, peb_h, y_h, ye_h):
        def chunk(k, _):
            base = sid * EPT + k * CM
            pltpu.sync_copy(col_h.at[pl.ds(base, CM)], colv)
            pltpu.sync_copy(row_h.at[pl.ds(base, CM)], rowv)
            dg0 = pltpu.async_copy(p2a_h.at[colv], g0, sem)
            dg1 = pltpu.async_copy(p2b_h.at[colv], g1, sem)
            dyg = pltpu.async_copy(y_h.at[colv], ygb, sem)
            pltpu.sync_copy(pea_h.at[pl.ds(base, CM)], w0b)
            pltpu.sync_copy(peb_h.at[pl.ds(base, CM)], w1b)
            pltpu.sync_copy(ye_h.at[pl.ds(base, CM)], yeb)
            dg0.wait()
            dg1.wait()

            def wmul(i, _):
                sl = pl.ds(i * 16, 16)
                w0b[sl] = w0b[sl] * g0[sl]
                w1b[sl] = w1b[sl] * g1[sl]
                return 0

            lax.fori_loop(0, CM // 16, wmul, 0)
            ds0 = pltpu.async_copy(w0b, sa_sh.at[rowv], sem, add=True)
            ds1 = pltpu.async_copy(w1b, sb_sh.at[rowv], sem, add=True)
            dyg.wait()

            def group(g, _):
                w0v = w0b[pl.ds(g * 16, 16)]
                w1v = w1b[pl.ds(g * 16, 16)]
                for j in range(16):
                    e = g * 16 + j
                    w0 = w0v[j]
                    w1 = w1v[j]
                    for t in range(4):
                        sl = pl.ds(t * 16, 16)
                        ygb[e, sl] = (ygb[e, sl] + yeb[e, sl]) * w0
                    for t in range(4, 8):
                        sl = pl.ds(t * 16, 16)
                        ygb[e, sl] = (ygb[e, sl] + yeb[e, sl]) * w1
                return 0

            lax.fori_loop(0, CM // 16, group, 0)
            pltpu.async_copy(ygb, z_sh.at[rowv], sem, add=True).wait()
            ds0.wait()
            ds1.wait()
            return 0

        lax.fori_loop(0, EPT // CM, chunk, 0)

    @pl.when(c == 0)
    def _():
        run(p20_h, p21_h, pe0_h, pe1_h, y0_h, ye0_h)

    @pl.when(c == 1)
    def _():
        run(p22_h, p23_h, pe2_h, pe3_h, y1_h, ye1_h)

    plsc.subcore_barrier()

    @pl.when(c == 0)
    def _():
        pltpu.sync_copy(sa_sh.at[pl.ds(sid * NR, NR)],
                        s0_h.at[pl.ds(sid * NR, NR)])
        pltpu.sync_copy(sb_sh.at[pl.ds(sid * NR, NR)],
                        s1_h.at[pl.ds(sid * NR, NR)])
        pltpu.sync_copy(z_sh.at[pl.ds(sid * NR, NR)],
                        z0_h.at[pl.ds(sid * NR, NR)])

    @pl.when(c == 1)
    def _():
        pltpu.sync_copy(sa_sh.at[pl.ds(sid * NR, NR)],
                        s2_h.at[pl.ds(sid * NR, NR)])
        pltpu.sync_copy(sb_sh.at[pl.ds(sid * NR, NR)],
                        s3_h.at[pl.ds(sid * NR, NR)])
        pltpu.sync_copy(z_sh.at[pl.ds(sid * NR, NR)],
                        z1_h.at[pl.ds(sid * NR, NR)])


def _c1_body(xw_ref, z0_ref, z1_ref, s0_ref, s1_ref, s2_ref, s3_ref, b1_ref,
             h_ref, st_ref):
    zcat = jnp.concatenate([z0_ref[...], z1_ref[...]], axis=1)
    rinvb = jnp.concatenate(
        [jnp.broadcast_to(1.0 / (s_ref[...] + 1e-16), (RT, OH))
         for s_ref in (s0_ref, s1_ref, s2_ref, s3_ref)], axis=1)
    ht = jax.nn.relu(xw_ref[...] + zcat * rinvb + b1_ref[...])
    h_ref[...] = ht
    st = jnp.stack([jnp.sum(ht, axis=0), jnp.sum(ht * ht, axis=0)])
    i = pl.program_id(0)

    @pl.when(i == 0)
    def _():
        st_ref[...] = st

    @pl.when(i > 0)
    def _():
        st_ref[...] += st


def _c1(xw, z0, z1, s0, s1, s2, s3, b1c):
    sspec = pl.BlockSpec((RT, 1), lambda i: (i, 0))
    return pl.pallas_call(
        _c1_body,
        grid=(N // RT,),
        in_specs=[
            pl.BlockSpec((RT, HOH), lambda i: (i, 0)),
            pl.BlockSpec((RT, 128), lambda i: (i, 0)),
            pl.BlockSpec((RT, 128), lambda i: (i, 0)),
            sspec, sspec, sspec, sspec,
            pl.BlockSpec((1, HOH), lambda i: (0, 0)),
        ],
        out_specs=[
            pl.BlockSpec((RT, HOH), lambda i: (i, 0)),
            pl.BlockSpec((2, HOH), lambda i: (0, 0)),
        ],
        out_shape=[
            jax.ShapeDtypeStruct((N, HOH), jnp.float32),
            jax.ShapeDtypeStruct((2, HOH), jnp.float32),
        ],
    )(xw, z0, z1, s0, s1, s2, s3, b1c)


def _c2_body(h_ref, st_ref, g_ref, bt_ref, b2_ref, w2_ref, out_ref):
    st = st_ref[...]
    mu = st[0:1] * (1.0 / N)
    var = st[1:2] * (1.0 / N) - mu * mu
    scale = g_ref[...] * jax.lax.rsqrt(var + 1e-5)
    w2eff = w2_ref[...] * scale.reshape(HOH, 1)
    bias = (jnp.dot(bt_ref[...] - mu * scale, w2_ref[...],
                    preferred_element_type=jnp.float32) + b2_ref[...])
    out_ref[...] = (jnp.dot(h_ref[...], w2eff,
                            preferred_element_type=jnp.float32) + bias)


def _c2(h, st, gc, bc, b2c, w2bd):
    return pl.pallas_call(
        _c2_body,
        grid=(N // RT,),
        in_specs=[
            pl.BlockSpec((RT, HOH), lambda i: (i, 0)),
            pl.BlockSpec((2, HOH), lambda i: (0, 0)),
            pl.BlockSpec((1, HOH), lambda i: (0, 0)),
            pl.BlockSpec((1, HOH), lambda i: (0, 0)),
            pl.BlockSpec((1, HOH), lambda i: (0, 0)),
            pl.BlockSpec((HOH, HOH), lambda i: (0, 0)),
        ],
        out_specs=pl.BlockSpec((RT, HOH), lambda i: (i, 0)),
        out_shape=jax.ShapeDtypeStruct((N, HOH), jnp.float32),
    )(h, st, gc, bc, b2c, w2bd)


def kernel(x, edge_index, edge_attr, a1_w, a2_w, ae_w, W1, b1, gamma, beta, W2, b2):
    row = edge_index[0]
    col = edge_index[1]
    a2t = jnp.zeros((D, HP), jnp.float32).at[:, :H].set(a2_w.T)
    aet = jnp.zeros((DE, HP), jnp.float32).at[:, :H].set(ae_w.T)
    w1x = W1[:, :D].transpose(1, 0, 2).reshape(D, HOH)
    w1o = W1[:, D:2 * D].transpose(1, 0, 2).reshape(D, HOH)
    w1e = W1[:, 2 * D:].transpose(1, 0, 2).reshape(DE, HOH)
    b1c = b1.reshape(1, HOH)
    gc = gamma.reshape(1, HOH)
    bc = beta.reshape(1, HOH)
    b2c = b2.reshape(1, HOH)
    w2bd = jax.scipy.linalg.block_diag(*[W2[i] for i in range(H)])
    zz1 = jnp.zeros((NP,), jnp.float32)
    zz128 = jnp.zeros((NP, 128), jnp.float32)

    p2, xw, y0, y1 = _a1(x, a2t, w1x, w1o)
    pe, ye0, ye1 = _a2(edge_attr, aet, w1e)
    pad = EP - E
    rowp = jnp.concatenate([row, jnp.zeros((pad,), jnp.int32)])
    colp = jnp.concatenate([col, jnp.zeros((pad,), jnp.int32)])
    pep = jnp.pad(pe, ((0, pad), (0, 0)))
    ye0p = jnp.pad(ye0, ((0, pad), (0, 0)))
    ye1p = jnp.pad(ye1, ((0, pad), (0, 0)))
    p2h = [p2[:, i] for i in range(H)]
    peh = [pep[:, i] for i in range(H)]

    s0, s1, s2, s3, z0, z1 = _sc(
        rowp, colp, p2h[0], p2h[1], p2h[2], p2h[3],
        peh[0], peh[1], peh[2], peh[3],
        y0, y1, ye0p, ye1p, zz1, zz128)

    h, st = _c1(xw, z0[:N], z1[:N],
                s0[:N].reshape(N, 1), s1[:N].reshape(N, 1),
                s2[:N].reshape(N, 1), s3[:N].reshape(N, 1), b1c)
    return _c2(h, st, gc, bc, b2c, w2bd)


# speedup vs baseline: 8.0532x; 1.3046x over previous
import functools
import math

import jax
import jax.numpy as jnp
from jax import lax
from jax.experimental import pallas as pl
from jax.experimental.pallas import tpu as pltpu
from jax.experimental.pallas import tpu_sc as plsc

N, E, D, DE, H, OH = 10000, 160000, 256, 16, 4, 64
HP = 16
HOH = H * OH
RT = 1000

NP = 10240
NR = NP // 16
CM = 128
EP = 161792
EPT = EP // 16
ET = 2048

_mesh = plsc.VectorSubcoreMesh(core_axis_name="c", subcore_axis_name="s",
                               num_cores=2, num_subcores=16)


def _a1_body(x_ref, a2t_ref, w1x_ref, w1o_ref, p2_ref, xw_ref, y0_ref, y1_ref):
    xt = x_ref[...]
    p2_ref[...] = jnp.exp(jnp.dot(xt, a2t_ref[...],
                                  preferred_element_type=jnp.float32)
                          * (1.0 / math.sqrt(OH)))
    xw_ref[...] = jnp.dot(xt, w1x_ref[...], preferred_element_type=jnp.float32)
    y = jnp.dot(xt, w1o_ref[...], preferred_element_type=jnp.float32)
    y0_ref[...] = y[:, :128]
    y1_ref[...] = y[:, 128:]


def _a1(x, a2t, w1x, w1o):
    return pl.pallas_call(
        _a1_body,
        grid=(N // RT,),
        in_specs=[
            pl.BlockSpec((RT, D), lambda i: (i, 0)),
            pl.BlockSpec((D, HP), lambda i: (0, 0)),
            pl.BlockSpec((D, HOH), lambda i: (0, 0)),
            pl.BlockSpec((D, HOH), lambda i: (0, 0)),
        ],
        out_specs=[
            pl.BlockSpec((RT, HP), lambda i: (i, 0)),
            pl.BlockSpec((RT, HOH), lambda i: (i, 0)),
            pl.BlockSpec((RT, 128), lambda i: (i, 0)),
            pl.BlockSpec((RT, 128), lambda i: (i, 0)),
        ],
        out_shape=[
            jax.ShapeDtypeStruct((N, HP), jnp.float32),
            jax.ShapeDtypeStruct((N, HOH), jnp.float32),
            jax.ShapeDtypeStruct((N, 128), jnp.float32),
            jax.ShapeDtypeStruct((N, 128), jnp.float32),
        ],
    )(x, a2t, w1x, w1o)


def _a2_body(ea_ref, aet_ref, w1e_ref, pet_ref, ye0_ref, ye1_ref):
    ea = ea_ref[...]
    pe = jnp.exp(jnp.dot(ea, aet_ref[...],
                         preferred_element_type=jnp.float32)
                 * (1.0 / math.sqrt(OH)))
    idx = (pl.program_id(0) * ET
           + jax.lax.broadcasted_iota(jnp.int32, (ET, HP), 0))
    pe = jnp.where(idx < E, pe, 0.0)
    pet_ref[...] = pe.T[:8]
    ye = jnp.dot(ea, w1e_ref[...], preferred_element_type=jnp.float32)
    ye0_ref[...] = ye[:, :128]
    ye1_ref[...] = ye[:, 128:]


def _a2(eap, aet, w1e):
    return pl.pallas_call(
        _a2_body,
        grid=(EP // ET,),
        in_specs=[
            pl.BlockSpec((ET, DE), lambda i: (i, 0)),
            pl.BlockSpec((DE, HP), lambda i: (0, 0)),
            pl.BlockSpec((DE, HOH), lambda i: (0, 0)),
        ],
        out_specs=[
            pl.BlockSpec((8, ET), lambda i: (0, i)),
            pl.BlockSpec((ET, 128), lambda i: (i, 0)),
            pl.BlockSpec((ET, 128), lambda i: (i, 0)),
        ],
        out_shape=[
            jax.ShapeDtypeStruct((8, EP), jnp.float32),
            jax.ShapeDtypeStruct((EP, 128), jnp.float32),
            jax.ShapeDtypeStruct((EP, 128), jnp.float32),
        ],
    )(eap, aet, w1e)


@functools.partial(
    pl.kernel, mesh=_mesh,
    out_type=[jax.ShapeDtypeStruct((NP,), jnp.float32),
              jax.ShapeDtypeStruct((NP,), jnp.float32),
              jax.ShapeDtypeStruct((NP,), jnp.float32),
              jax.ShapeDtypeStruct((NP,), jnp.float32),
              jax.ShapeDtypeStruct((NP, 128), jnp.float32),
              jax.ShapeDtypeStruct((NP, 128), jnp.float32)],
    scratch_types=[pltpu.VMEM((CM,), jnp.int32),
                   pltpu.VMEM((CM,), jnp.int32),
                   pltpu.VMEM((CM,), jnp.float32),
                   pltpu.VMEM((CM,), jnp.float32),
                   pltpu.VMEM((CM,), jnp.float32),
                   pltpu.VMEM((CM,), jnp.float32),
                   pltpu.VMEM((CM, 128), jnp.float32),
                   pltpu.VMEM((CM, 128), jnp.float32),
                   pltpu.VMEM_SHARED((NP,), jnp.float32),
                   pltpu.VMEM_SHARED((NP,), jnp.float32),
                   pltpu.VMEM_SHARED((NP, 128), jnp.float32),
                   pltpu.SemaphoreType.DMA])
def _sc(row_h, col_h, p20_h, p21_h, p22_h, p23_h, pe0_h, pe1_h, pe2_h, pe3_h,
        y0_h, y1_h, ye0_h, ye1_h, zz1_h, zz128_h,
        s0_h, s1_h, s2_h, s3_h, z0_h, z1_h,
        colv, rowv, g0, g1, w0b, w1b, ygb, yeb,
        sa_sh, sb_sh, z_sh, sem):
    c = lax.axis_index("c")
    sid = lax.axis_index("s")
    sl_n = pl.ds(sid * NR, NR)
    pltpu.sync_copy(zz1_h.at[sl_n], sa_sh.at[sl_n])
    pltpu.sync_copy(zz1_h.at[sl_n], sb_sh.at[sl_n])
    pltpu.sync_copy(zz128_h.at[sl_n], z_sh.at[sl_n])
    plsc.subcore_barrier()

    def run(p2a_h, p2b_h, pea_h, peb_h, y_h, ye_h):
        def chunk(k, _):
            base = sid * EPT + k * CM
            pltpu.sync_copy(col_h.at[pl.ds(base, CM)], colv)
            pltpu.sync_copy(row_h.at[pl.ds(base, CM)], rowv)
            dg0 = pltpu.async_copy(p2a_h.at[colv], g0, sem)
            dg1 = pltpu.async_copy(p2b_h.at[colv], g1, sem)
            dyg = pltpu.async_copy(y_h.at[colv], ygb, sem)
            pltpu.sync_copy(pea_h.at[pl.ds(base, CM)], w0b)
            pltpu.sync_copy(peb_h.at[pl.ds(base, CM)], w1b)
            pltpu.sync_copy(ye_h.at[pl.ds(base, CM)], yeb)
            dg0.wait()
            dg1.wait()

            def wmul(i, _):
                sl = pl.ds(i * 16, 16)
                w0b[sl] = w0b[sl] * g0[sl]
                w1b[sl] = w1b[sl] * g1[sl]
                return 0

            lax.fori_loop(0, CM // 16, wmul, 0)
            ds0 = pltpu.async_copy(w0b, sa_sh.at[rowv], sem, add=True)
            ds1 = pltpu.async_copy(w1b, sb_sh.at[rowv], sem, add=True)
            dyg.wait()

            def group(g, _):
                w0v = w0b[pl.ds(g * 16, 16)]
                w1v = w1b[pl.ds(g * 16, 16)]
                for j in range(16):
                    e = g * 16 + j
                    w0 = w0v[j]
                    w1 = w1v[j]
                    for t in range(4):
                        sl = pl.ds(t * 16, 16)
                        ygb[e, sl] = (ygb[e, sl] + yeb[e, sl]) * w0
                    for t in range(4, 8):
                        sl = pl.ds(t * 16, 16)
                        ygb[e, sl] = (ygb[e, sl] + yeb[e, sl]) * w1
                return 0

            lax.fori_loop(0, CM // 16, group, 0)
            pltpu.async_copy(ygb, z_sh.at[rowv], sem, add=True).wait()
            ds0.wait()
            ds1.wait()
            return 0

        lax.fori_loop(0, EPT // CM, chunk, 0)

    @pl.when(c == 0)
    def _():
        run(p20_h, p21_h, pe0_h, pe1_h, y0_h, ye0_h)

    @pl.when(c == 1)
    def _():
        run(p22_h, p23_h, pe2_h, pe3_h, y1_h, ye1_h)

    plsc.subcore_barrier()

    @pl.when(c == 0)
    def _():
        pltpu.sync_copy(sa_sh.at[sl_n], s0_h.at[sl_n])
        pltpu.sync_copy(sb_sh.at[sl_n], s1_h.at[sl_n])
        pltpu.sync_copy(z_sh.at[sl_n], z0_h.at[sl_n])

    @pl.when(c == 1)
    def _():
        pltpu.sync_copy(sa_sh.at[sl_n], s2_h.at[sl_n])
        pltpu.sync_copy(sb_sh.at[sl_n], s3_h.at[sl_n])
        pltpu.sync_copy(z_sh.at[sl_n], z1_h.at[sl_n])


def _c1_body(xw_ref, z0_ref, z1_ref,
             s0_ref, s1_ref, s2_ref, s3_ref, b1_ref, h_ref, st_ref):
    zcat = jnp.concatenate([z0_ref[...], z1_ref[...]], axis=1)
    rinvb = jnp.concatenate(
        [jnp.broadcast_to(1.0 / (s_ref[...] + 1e-16), (RT, OH))
         for s_ref in (s0_ref, s1_ref, s2_ref, s3_ref)], axis=1)
    ht = jax.nn.relu(xw_ref[...] + zcat * rinvb + b1_ref[...])
    h_ref[...] = ht
    st = jnp.stack([jnp.sum(ht, axis=0), jnp.sum(ht * ht, axis=0)])
    i = pl.program_id(0)

    @pl.when(i == 0)
    def _():
        st_ref[...] = st

    @pl.when(i > 0)
    def _():
        st_ref[...] += st


def _c1(xw, z0, z1, s0, s1, s2, s3, b1c):
    sspec = pl.BlockSpec((RT, 1), lambda i: (i, 0))
    return pl.pallas_call(
        _c1_body,
        grid=(N // RT,),
        in_specs=[
            pl.BlockSpec((RT, HOH), lambda i: (i, 0)),
            pl.BlockSpec((RT, 128), lambda i: (i, 0)),
            pl.BlockSpec((RT, 128), lambda i: (i, 0)),
            sspec, sspec, sspec, sspec,
            pl.BlockSpec((1, HOH), lambda i: (0, 0)),
        ],
        out_specs=[
            pl.BlockSpec((RT, HOH), lambda i: (i, 0)),
            pl.BlockSpec((2, HOH), lambda i: (0, 0)),
        ],
        out_shape=[
            jax.ShapeDtypeStruct((N, HOH), jnp.float32),
            jax.ShapeDtypeStruct((2, HOH), jnp.float32),
        ],
    )(xw, z0, z1, s0, s1, s2, s3, b1c)


def _c2_body(h_ref, st_ref, g_ref, bt_ref, b2_ref, w2_ref, out_ref):
    st = st_ref[...]
    mu = st[0:1] * (1.0 / N)
    var = st[1:2] * (1.0 / N) - mu * mu
    scale = g_ref[...] * jax.lax.rsqrt(var + 1e-5)
    w2eff = w2_ref[...] * scale.reshape(HOH, 1)
    bias = (jnp.dot(bt_ref[...] - mu * scale, w2_ref[...],
                    preferred_element_type=jnp.float32) + b2_ref[...])
    out_ref[...] = (jnp.dot(h_ref[...], w2eff,
                            preferred_element_type=jnp.float32) + bias)


def _c2(h, st, gc, bc, b2c, w2bd):
    return pl.pallas_call(
        _c2_body,
        grid=(N // RT,),
        in_specs=[
            pl.BlockSpec((RT, HOH), lambda i: (i, 0)),
            pl.BlockSpec((2, HOH), lambda i: (0, 0)),
            pl.BlockSpec((1, HOH), lambda i: (0, 0)),
            pl.BlockSpec((1, HOH), lambda i: (0, 0)),
            pl.BlockSpec((1, HOH), lambda i: (0, 0)),
            pl.BlockSpec((HOH, HOH), lambda i: (0, 0)),
        ],
        out_specs=pl.BlockSpec((RT, HOH), lambda i: (i, 0)),
        out_shape=jax.ShapeDtypeStruct((N, HOH), jnp.float32),
    )(h, st, gc, bc, b2c, w2bd)


def kernel(x, edge_index, edge_attr, a1_w, a2_w, ae_w, W1, b1, gamma, beta, W2, b2):
    row = edge_index[0]
    col = edge_index[1]
    a2t = jnp.zeros((D, HP), jnp.float32).at[:, :H].set(a2_w.T)
    aet = jnp.zeros((DE, HP), jnp.float32).at[:, :H].set(ae_w.T)
    w1x = W1[:, :D].transpose(1, 0, 2).reshape(D, HOH)
    w1o = W1[:, D:2 * D].transpose(1, 0, 2).reshape(D, HOH)
    w1e = W1[:, 2 * D:].transpose(1, 0, 2).reshape(DE, HOH)
    b1c = b1.reshape(1, HOH)
    gc = gamma.reshape(1, HOH)
    bc = beta.reshape(1, HOH)
    b2c = b2.reshape(1, HOH)
    w2bd = jax.scipy.linalg.block_diag(*[W2[i] for i in range(H)])
    zz1 = jnp.zeros((NP,), jnp.float32)
    zz128 = jnp.zeros((NP, 128), jnp.float32)

    pad = EP - E
    rowp = jnp.concatenate([row, jnp.zeros((pad,), jnp.int32)])
    colp = jnp.concatenate([col, jnp.zeros((pad,), jnp.int32)])
    eap = jnp.pad(edge_attr, ((0, pad), (0, 0)))

    p2, xw, y0, y1 = _a1(x, a2t, w1x, w1o)
    pet, ye0, ye1 = _a2(eap, aet, w1e)
    p2h = [p2[:, i] for i in range(H)]
    peh = [pet[i] for i in range(H)]

    s0, s1, s2, s3, z0, z1 = _sc(
        rowp, colp, p2h[0], p2h[1], p2h[2], p2h[3],
        peh[0], peh[1], peh[2], peh[3],
        y0, y1, ye0, ye1, zz1, zz128)

    h, st = _c1(xw, z0[:N], z1[:N],
                s0[:N].reshape(N, 1), s1[:N].reshape(N, 1),
                s2[:N].reshape(N, 1), s3[:N].reshape(N, 1), b1c)
    return _c2(h, st, gc, bc, b2c, w2bd)
